# baseline (speedup 1.0000x reference)
import jax
import jax.numpy as jnp
from jax import lax
from jax.experimental import pallas as pl

NB = 8192
MB = 1024


def _chamfer_body(x1m2_ref, x2t_ref, out1_ref, out2_ref):
    i_n = pl.program_id(1)
    n_last = pl.num_programs(1) - 1
    m_total = x2t_ref.shape[2]
    x1m2 = x1m2_ref[0]
    sq1 = 0.25 * jnp.sum(x1m2 * x1m2, axis=1, keepdims=True)
    acc1 = None
    for j in range(m_total // MB):
        x2 = x2t_ref[0, :, j * MB:(j + 1) * MB]
        sq2 = jnp.sum(x2 * x2, axis=0, keepdims=True)
        cross = lax.dot_general(
            x1m2, x2, (((1,), (0,)), ((), ())),
            preferred_element_type=jnp.float32)
        g = cross + sq2
        pm1 = jnp.min(g, axis=1)
        acc1 = pm1 if acc1 is None else jnp.minimum(acc1, pm1)
        f = cross + sq1
        pm2 = jnp.min(f, axis=0)
        sl = pl.ds(j * MB, MB)

        @pl.when(i_n == 0)
        def _init():
            out2_ref[0, 0, sl] = pm2

        @pl.when(jnp.logical_and(i_n != 0, i_n != n_last))
        def _acc():
            out2_ref[0, 0, sl] = jnp.minimum(out2_ref[0, 0, sl], pm2)

        @pl.when(jnp.logical_and(i_n != 0, i_n == n_last))
        def _fin():
            out2_ref[0, 0, sl] = (
                jnp.minimum(out2_ref[0, 0, sl], pm2) + sq2[0, :])

    out1_ref[0, 0, 0, :] = acc1 + sq1[:, 0]


def kernel(xyz1, xyz2):
    b, n, _ = xyz1.shape
    m = xyz2.shape[1]
    x1m2 = -2.0 * xyz1
    x2t = jnp.transpose(xyz2, (0, 2, 1))
    dist1, dist2 = pl.pallas_call(
        _chamfer_body,
        grid=(b, n // NB),
        in_specs=[
            pl.BlockSpec((1, NB, 3), lambda bb, ii: (bb, ii, 0)),
            pl.BlockSpec((1, 3, m), lambda bb, ii: (bb, 0, 0)),
        ],
        out_specs=[
            pl.BlockSpec((1, 1, 1, NB), lambda bb, ii: (bb, ii, 0, 0)),
            pl.BlockSpec((1, 1, m), lambda bb, ii: (bb, 0, 0)),
        ],
        out_shape=[
            jax.ShapeDtypeStruct((b, n // NB, 1, NB), jnp.float32),
            jax.ShapeDtypeStruct((b, 1, m), jnp.float32),
        ],
    )(x1m2, x2t)
    return (dist1.reshape(b, n), dist2.reshape(b, m))


# speedup vs baseline: 1.4268x; 1.4268x over previous
import jax
import jax.numpy as jnp
from jax import lax
from jax.experimental import pallas as pl

NB = 4096
MB = 4096


def _chamfer_body(x1m2_ref, x2t_ref, out1_ref, out2_ref):
    i_n = pl.program_id(1)
    n_last = pl.num_programs(1) - 1
    m_total = x2t_ref.shape[2]
    x1m2 = x1m2_ref[0]
    sq1 = 0.25 * jnp.sum(x1m2 * x1m2, axis=1, keepdims=True)
    acc1 = None
    for j in range(m_total // MB):
        x2 = x2t_ref[0, :, j * MB:(j + 1) * MB]
        sq2 = jnp.sum(x2 * x2, axis=0, keepdims=True)
        cross = lax.dot_general(
            x1m2, x2, (((1,), (0,)), ((), ())),
            preferred_element_type=jnp.float32)
        g = cross + sq2
        pm1 = jnp.min(g, axis=1)
        acc1 = pm1 if acc1 is None else jnp.minimum(acc1, pm1)
        f = cross + sq1
        pm2 = jnp.min(f, axis=0)
        sl = pl.ds(j * MB, MB)

        @pl.when(i_n == 0)
        def _init():
            out2_ref[0, 0, sl] = pm2

        @pl.when(jnp.logical_and(i_n != 0, i_n != n_last))
        def _acc():
            out2_ref[0, 0, sl] = jnp.minimum(out2_ref[0, 0, sl], pm2)

        @pl.when(jnp.logical_and(i_n != 0, i_n == n_last))
        def _fin():
            out2_ref[0, 0, sl] = (
                jnp.minimum(out2_ref[0, 0, sl], pm2) + sq2[0, :])

    out1_ref[0, 0, 0, :] = acc1 + sq1[:, 0]


def kernel(xyz1, xyz2):
    b, n, _ = xyz1.shape
    m = xyz2.shape[1]
    x1m2 = -2.0 * xyz1
    x2t = jnp.transpose(xyz2, (0, 2, 1))
    dist1, dist2 = pl.pallas_call(
        _chamfer_body,
        grid=(b, n // NB),
        in_specs=[
            pl.BlockSpec((1, NB, 3), lambda bb, ii: (bb, ii, 0)),
            pl.BlockSpec((1, 3, m), lambda bb, ii: (bb, 0, 0)),
        ],
        out_specs=[
            pl.BlockSpec((1, 1, 1, NB), lambda bb, ii: (bb, ii, 0, 0)),
            pl.BlockSpec((1, 1, m), lambda bb, ii: (bb, 0, 0)),
        ],
        out_shape=[
            jax.ShapeDtypeStruct((b, n // NB, 1, NB), jnp.float32),
            jax.ShapeDtypeStruct((b, 1, m), jnp.float32),
        ],
    )(x1m2, x2t)
    return (dist1.reshape(b, n), dist2.reshape(b, m))


# speedup vs baseline: 1.6766x; 1.1751x over previous
import jax
import jax.numpy as jnp
from jax import lax
from jax.experimental import pallas as pl

NB = 4096
MB = 4096


def _split3(v):
    hi = v.astype(jnp.bfloat16).astype(jnp.float32)
    r = v - hi
    mid = r.astype(jnp.bfloat16).astype(jnp.float32)
    lo = (r - mid).astype(jnp.bfloat16).astype(jnp.float32)
    return hi, mid, lo


def _chamfer_body(a_ref, bt_ref, out1_ref, out2_ref):
    i_n = pl.program_id(1)
    m_total = bt_ref.shape[2]
    a = a_ref[0]
    acc1 = None
    for j in range(m_total // MB):
        b2 = bt_ref[0, :, j * MB:(j + 1) * MB]
        d = lax.dot_general(
            a, b2, (((1,), (0,)), ((), ())),
            preferred_element_type=jnp.float32)
        pm1 = jnp.min(d, axis=1)
        acc1 = pm1 if acc1 is None else jnp.minimum(acc1, pm1)
        pm2 = jnp.min(d, axis=0)
        sl = pl.ds(j * MB, MB)

        @pl.when(i_n == 0)
        def _init():
            out2_ref[0, 0, sl] = pm2

        @pl.when(i_n != 0)
        def _acc():
            out2_ref[0, 0, sl] = jnp.minimum(out2_ref[0, 0, sl], pm2)

    out1_ref[0, 0, 0, :] = acc1


def _augment(xyz1, xyz2):
    b, n, _ = xyz1.shape
    m = xyz2.shape[1]
    sq1 = jnp.sum(xyz1 * xyz1, axis=-1)
    sq2 = jnp.sum(xyz2 * xyz2, axis=-1)
    s1h, s1m, s1l = _split3(sq1)
    s2h, s2m, s2l = _split3(sq2)
    ones_n = jnp.ones((b, n, 1), jnp.float32)
    a_aug = jnp.concatenate(
        [-2.0 * xyz1, s1h[..., None], s1m[..., None], s1l[..., None],
         ones_n, ones_n, ones_n], axis=-1)
    x2t = jnp.transpose(xyz2, (0, 2, 1))
    ones_m = jnp.ones((b, 1, m), jnp.float32)
    b_aug = jnp.concatenate(
        [x2t, ones_m, ones_m, ones_m,
         s2h[:, None, :], s2m[:, None, :], s2l[:, None, :]],
        axis=1)
    return a_aug, b_aug


def kernel(xyz1, xyz2):
    b, n, _ = xyz1.shape
    m = xyz2.shape[1]
    a_aug, b_aug = _augment(xyz1, xyz2)
    dist1, dist2 = pl.pallas_call(
        _chamfer_body,
        grid=(b, n // NB),
        in_specs=[
            pl.BlockSpec((1, NB, 9), lambda bb, ii: (bb, ii, 0)),
            pl.BlockSpec((1, 9, m), lambda bb, ii: (bb, 0, 0)),
        ],
        out_specs=[
            pl.BlockSpec((1, 1, 1, NB), lambda bb, ii: (bb, ii, 0, 0)),
            pl.BlockSpec((1, 1, m), lambda bb, ii: (bb, 0, 0)),
        ],
        out_shape=[
            jax.ShapeDtypeStruct((b, n // NB, 1, NB), jnp.float32),
            jax.ShapeDtypeStruct((b, 1, m), jnp.float32),
        ],
    )(a_aug, b_aug)
    return (dist1.reshape(b, n), dist2.reshape(b, m))
